# baseline (speedup 1.0000x reference)
import functools

import jax
import jax.numpy as jnp
import numpy as np
from jax import lax
from jax.experimental import pallas as pl
from jax.experimental.pallas import tpu as pltpu
from jax.experimental.pallas import tpu_sc as plsc

K = 128
TAU = 16
OCS = 16
NTOK = 4096
D = K * TAU
V = K * (2 ** TAU)

BT = 512

NW = 32
TOK_PER_W = NTOK // NW
TB = 16
NBATCH = TOK_PER_W // TB
TBK = TB * K


def _prep_body(x_ref, wc_ref, ws_ref, codes_ref, p_ref):
    xb = x_ref[...]
    bits = (xb >= 0.0).astype(jnp.float32)
    z = 2.0 * xb
    ls = jnp.minimum(z, 0.0) - jnp.log(1.0 + jnp.exp(-jnp.abs(z)))
    cf = lax.dot_general(bits, wc_ref[...], (((1,), (0,)), ((), ())),
                         preferred_element_type=jnp.float32)
    lp = lax.dot_general(ls, ws_ref[...], (((1,), (0,)), ((), ())),
                         preferred_element_type=jnp.float32)
    chunk = lax.broadcasted_iota(jnp.int32, cf.shape, 1)
    codes_ref[...] = cf.astype(jnp.int32) + chunk * (2 ** TAU)
    p_ref[...] = jnp.exp(lp)


def _prep(x2d):
    wc = np.zeros((D, K), dtype=np.float32)
    ws = np.zeros((D, K), dtype=np.float32)
    for k in range(K):
        for j in range(TAU):
            wc[k * TAU + j, k] = float(2 ** (TAU - 1 - j))
            ws[k * TAU + j, k] = 1.0
    grid = (NTOK // BT,)
    return pl.pallas_call(
        _prep_body,
        grid=grid,
        in_specs=[
            pl.BlockSpec((BT, D), lambda i: (i, 0)),
            pl.BlockSpec((D, K), lambda i: (0, 0)),
            pl.BlockSpec((D, K), lambda i: (0, 0)),
        ],
        out_specs=[
            pl.BlockSpec((BT, K), lambda i: (i, 0)),
            pl.BlockSpec((BT, K), lambda i: (i, 0)),
        ],
        out_shape=[
            jax.ShapeDtypeStruct((NTOK, K), jnp.int32),
            jax.ShapeDtypeStruct((NTOK, K), jnp.float32),
        ],
    )(x2d, jnp.asarray(wc), jnp.asarray(ws))


def _gather_scale_body(tables_hbm, codes_hbm, p_hbm, out_hbm,
                       idx_v, p_v, rows_v, sem):
    wid = lax.axis_index("s") * 2 + lax.axis_index("c")
    tok0 = wid * TOK_PER_W
    iota16 = lax.iota(jnp.int32, 16)
    for b in range(NBATCH):
        tb0 = tok0 + b * TB
        pltpu.sync_copy(codes_hbm.at[pl.ds(tb0, TB)], idx_v)
        pltpu.sync_copy(p_hbm.at[pl.ds(tb0, TB)], p_v)
        cps = [
            pltpu.async_copy(tables_hbm.at[idx_v.at[g]],
                             rows_v.at[pl.ds(g * K, K)], sem)
            for g in range(TB)
        ]
        for cp in cps:
            cp.wait()

        def scale(j, _):
            g = j // 8
            c0 = (j % 8) * 16
            pv16 = p_v[g, pl.ds(c0, 16)]
            for jj in range(16):
                r = j * 16 + jj
                psplat = jnp.full((16,), pv16[jj], jnp.float32)
                rows_v[r, :] = rows_v[r, :] * psplat
            return 0

        lax.fori_loop(0, TBK // 16, scale, 0)
        pltpu.sync_copy(rows_v, out_hbm.at[pl.ds(tb0 * K, TBK)])


def _gather_scale(tables_flat, codes, p):
    mesh = plsc.VectorSubcoreMesh(core_axis_name="c", subcore_axis_name="s")
    f = functools.partial(
        pl.kernel,
        mesh=mesh,
        compiler_params=pltpu.CompilerParams(use_tc_tiling_on_sc=False),
        out_type=jax.ShapeDtypeStruct((NTOK * K, OCS), jnp.float32),
        scratch_types=[
            pltpu.VMEM((TB, K), jnp.int32),
            pltpu.VMEM((TB, K), jnp.float32),
            pltpu.VMEM((TBK, OCS), jnp.float32),
            pltpu.SemaphoreType.DMA,
        ],
    )(_gather_scale_body)
    return f(tables_flat, codes, p)


def kernel(x, tables):
    B, S, _ = x.shape
    x2d = x.reshape(NTOK, D)
    codes, p = _prep(x2d)
    tables_flat = tables.reshape(V, OCS)
    out = _gather_scale(tables_flat, codes, p)
    return out.reshape(B, S, K * OCS)


# speedup vs baseline: 5.7013x; 5.7013x over previous
import functools

import jax
import jax.numpy as jnp
import numpy as np
from jax import lax
from jax.experimental import pallas as pl
from jax.experimental.pallas import tpu as pltpu
from jax.experimental.pallas import tpu_sc as plsc

K = 128
TAU = 16
OCS = 16
NTOK = 4096
D = K * TAU
V = K * (2 ** TAU)

BT = 512
VG = 256

NW = 32
TOK_PER_W = NTOK // NW
TB = 16
NBATCH = TOK_PER_W // TB
TBK = TB * K


def _shuffle_body(a2_ref, out_ref):
    x = a2_ref[...]
    t = jnp.transpose(x, (1, 0))
    t4 = t.reshape(32, 8, 16, 128)
    t5 = jnp.transpose(t4, (0, 2, 1, 3))
    out_ref[...] = t5.reshape(4096, 128)


def _shuffle(a2):
    return pl.pallas_call(
        _shuffle_body,
        grid=(65536 // VG,),
        in_specs=[pl.BlockSpec((D, VG), lambda g: (0, g))],
        out_specs=pl.BlockSpec((16 * VG, 128), lambda g: (g, 0)),
        out_shape=jax.ShapeDtypeStruct((V // 8, 128), jnp.float32),
    )(a2)


def _prep_body(x_ref, wc_ref, ws_ref, codes_ref, p_ref):
    xb = x_ref[...]
    bits = (xb >= 0.0).astype(jnp.float32)
    z = 2.0 * xb
    ls = jnp.minimum(z, 0.0) - jnp.log(1.0 + jnp.exp(-jnp.abs(z)))
    cf = lax.dot_general(bits, wc_ref[...], (((1,), (0,)), ((), ())),
                         preferred_element_type=jnp.float32)
    lp = lax.dot_general(ls, ws_ref[...], (((1,), (0,)), ((), ())),
                         preferred_element_type=jnp.float32)
    h = cf.astype(jnp.int32)
    kcol = lax.broadcasted_iota(jnp.int32, h.shape, 1)
    codes_ref[...] = ((h >> 3) * 1024 + (h & 7) * 8
                      + (kcol >> 3) * 64 + (kcol & 7))
    p_ref[...] = jnp.exp(lp)


def _prep(x2d):
    wc = np.zeros((D, K), dtype=np.float32)
    ws = np.zeros((D, K), dtype=np.float32)
    for k in range(K):
        for j in range(TAU):
            wc[k * TAU + j, k] = float(2 ** (TAU - 1 - j))
            ws[k * TAU + j, k] = 1.0
    grid = (NTOK // BT,)
    return pl.pallas_call(
        _prep_body,
        grid=grid,
        in_specs=[
            pl.BlockSpec((BT, D), lambda i: (i, 0)),
            pl.BlockSpec((D, K), lambda i: (0, 0)),
            pl.BlockSpec((D, K), lambda i: (0, 0)),
        ],
        out_specs=[
            pl.BlockSpec((BT, K), lambda i: (i, 0)),
            pl.BlockSpec((BT, K), lambda i: (i, 0)),
        ],
        out_shape=[
            jax.ShapeDtypeStruct((NTOK, K), jnp.int32),
            jax.ShapeDtypeStruct((NTOK, K), jnp.float32),
        ],
    )(x2d, jnp.asarray(wc), jnp.asarray(ws))


def _gather_scale_body(tables_hbm, codes_hbm, p_hbm, out_hbm,
                       idx_v, p_v, rows_v, sem):
    wid = lax.axis_index("s") * 2 + lax.axis_index("c")
    tok0 = wid * TOK_PER_W
    for b in range(NBATCH):
        tb0 = tok0 + b * TB
        pltpu.sync_copy(codes_hbm.at[pl.ds(tb0, TB)], idx_v)
        pltpu.sync_copy(p_hbm.at[pl.ds(tb0, TB)], p_v)
        cps = [
            pltpu.async_copy(tables_hbm.at[idx_v.at[g]],
                             rows_v.at[pl.ds(g * K, K)], sem)
            for g in range(TB)
        ]
        for cp in cps:
            cp.wait()

        def scale(j, _):
            g = j // 8
            c0 = (j % 8) * 16
            pv16 = p_v[g, pl.ds(c0, 16)]
            for jj in range(16):
                r = j * 16 + jj
                psplat = jnp.full((16,), pv16[jj], jnp.float32)
                rows_v[r, :] = rows_v[r, :] * psplat
            return 0

        lax.fori_loop(0, TBK // 16, scale, 0)
        pltpu.sync_copy(rows_v, out_hbm.at[pl.ds(tb0 * K, TBK)])


def _gather_scale(tables_flat, codes, p):
    mesh = plsc.VectorSubcoreMesh(core_axis_name="c", subcore_axis_name="s")
    f = functools.partial(
        pl.kernel,
        mesh=mesh,
        compiler_params=pltpu.CompilerParams(use_tc_tiling_on_sc=False),
        out_type=jax.ShapeDtypeStruct((NTOK * K, OCS), jnp.float32),
        scratch_types=[
            pltpu.VMEM((TB, K), jnp.int32),
            pltpu.VMEM((TB, K), jnp.float32),
            pltpu.VMEM((TBK, OCS), jnp.float32),
            pltpu.SemaphoreType.DMA,
        ],
    )(_gather_scale_body)
    return f(tables_flat, codes, p)


def kernel(x, tables):
    B, S, _ = x.shape
    x2d = x.reshape(NTOK, D)
    a2 = jnp.transpose(tables, (0, 2, 1)).reshape(D, 2 ** TAU)
    g = _shuffle(a2)
    lin = g.reshape(V, OCS)
    codes, p = _prep(x2d)
    out = _gather_scale(lin, codes, p)
    return out.reshape(B, S, K * OCS)


# speedup vs baseline: 6.3352x; 1.1112x over previous
import functools

import jax
import jax.numpy as jnp
import numpy as np
from jax import lax
from jax.experimental import pallas as pl
from jax.experimental.pallas import tpu as pltpu
from jax.experimental.pallas import tpu_sc as plsc

K = 128
TAU = 16
OCS = 16
NTOK = 4096
D = K * TAU
V = K * (2 ** TAU)

BT = 512
VG = 512

NW = 32
TOK_PER_W = NTOK // NW
TB = 16
NBATCH = TOK_PER_W // TB
TBK = TB * K


def _shuffle_body(a2_ref, out_ref):
    x = a2_ref[...]
    t = jnp.transpose(x, (1, 0))
    t4 = t.reshape(VG // 8, 8, 16, 128)
    t5 = jnp.transpose(t4, (0, 2, 1, 3))
    out_ref[...] = t5.reshape(16 * VG, 128)


def _shuffle(a2):
    return pl.pallas_call(
        _shuffle_body,
        grid=(65536 // VG,),
        in_specs=[pl.BlockSpec((D, VG), lambda g: (0, g))],
        out_specs=pl.BlockSpec((16 * VG, 128), lambda g: (g, 0)),
        out_shape=jax.ShapeDtypeStruct((V // 8, 128), jnp.float32),
    )(a2)


def _prep_body(x_ref, wc_ref, ws_ref, codes_ref, p_ref):
    xb = x_ref[...]
    bits = (xb >= 0.0).astype(jnp.float32)
    z = 2.0 * xb
    ls = jnp.minimum(z, 0.0) - jnp.log(1.0 + jnp.exp(-jnp.abs(z)))
    cf = lax.dot_general(bits, wc_ref[...], (((1,), (0,)), ((), ())),
                         preferred_element_type=jnp.float32)
    lp = lax.dot_general(ls, ws_ref[...], (((1,), (0,)), ((), ())),
                         preferred_element_type=jnp.float32)
    h = cf.astype(jnp.int32)
    kcol = lax.broadcasted_iota(jnp.int32, h.shape, 1)
    codes_ref[...] = ((h >> 3) * 1024 + (h & 7) * 8
                      + (kcol >> 3) * 64 + (kcol & 7))
    p_ref[...] = jnp.exp(lp)


def _prep(x2d):
    wc = np.zeros((D, K), dtype=np.float32)
    ws = np.zeros((D, K), dtype=np.float32)
    for k in range(K):
        for j in range(TAU):
            wc[k * TAU + j, k] = float(2 ** (TAU - 1 - j))
            ws[k * TAU + j, k] = 1.0
    grid = (NTOK // BT,)
    return pl.pallas_call(
        _prep_body,
        grid=grid,
        in_specs=[
            pl.BlockSpec((BT, D), lambda i: (i, 0)),
            pl.BlockSpec((D, K), lambda i: (0, 0)),
            pl.BlockSpec((D, K), lambda i: (0, 0)),
        ],
        out_specs=[
            pl.BlockSpec((BT, K), lambda i: (i, 0)),
            pl.BlockSpec((BT, K), lambda i: (i, 0)),
        ],
        out_shape=[
            jax.ShapeDtypeStruct((NTOK, K), jnp.int32),
            jax.ShapeDtypeStruct((NTOK, K), jnp.float32),
        ],
    )(x2d, jnp.asarray(wc), jnp.asarray(ws))


def _gather_scale_body(tables_hbm, codes_hbm, p_hbm, out_hbm,
                       idx_v, p_v, rows_v, sem):
    wid = lax.axis_index("s") * 2 + lax.axis_index("c")
    tok0 = wid * TOK_PER_W
    for b in range(NBATCH):
        tb0 = tok0 + b * TB
        pltpu.sync_copy(codes_hbm.at[pl.ds(tb0, TB)], idx_v)
        pltpu.sync_copy(p_hbm.at[pl.ds(tb0, TB)], p_v)
        cps = [
            pltpu.async_copy(tables_hbm.at[idx_v.at[g]],
                             rows_v.at[pl.ds(g * K, K)], sem)
            for g in range(TB)
        ]
        for cp in cps:
            cp.wait()

        def scale(j, _):
            g = j // 8
            c0 = (j % 8) * 16
            pv16 = p_v[g, pl.ds(c0, 16)]
            for jj in range(16):
                r = j * 16 + jj
                psplat = jnp.full((16,), pv16[jj], jnp.float32)
                rows_v[r, :] = rows_v[r, :] * psplat
            return 0

        lax.fori_loop(0, TBK // 16, scale, 0)
        pltpu.sync_copy(rows_v, out_hbm.at[pl.ds(tb0 * K, TBK)])


def _gather_scale(tables_flat, codes, p):
    mesh = plsc.VectorSubcoreMesh(core_axis_name="c", subcore_axis_name="s")
    f = functools.partial(
        pl.kernel,
        mesh=mesh,
        compiler_params=pltpu.CompilerParams(use_tc_tiling_on_sc=False),
        out_type=jax.ShapeDtypeStruct((NTOK * K, OCS), jnp.float32),
        scratch_types=[
            pltpu.VMEM((TB, K), jnp.int32),
            pltpu.VMEM((TB, K), jnp.float32),
            pltpu.VMEM((TBK, OCS), jnp.float32),
            pltpu.SemaphoreType.DMA,
        ],
    )(_gather_scale_body)
    return f(tables_flat, codes, p)


def kernel(x, tables):
    B, S, _ = x.shape
    x2d = x.reshape(NTOK, D)
    a2 = jnp.transpose(tables, (0, 2, 1)).reshape(D, 2 ** TAU)
    g = _shuffle(a2)
    lin = g.reshape(V, OCS)
    codes, p = _prep(x2d)
    out = _gather_scale(lin, codes, p)
    return out.reshape(B, S, K * OCS)
